# Initial kernel scaffold; baseline (speedup 1.0000x reference)
import jax
import jax.numpy as jnp
from jax.experimental import pallas as pl


def kernel(x, edge_index, W1, b1, W2, b2):
    raise NotImplementedError("write your pallas kernel here")


# speedup vs baseline: 17.7473x; 17.7473x over previous
import functools

import jax
import jax.numpy as jnp
from jax import lax
from jax.experimental import pallas as pl
from jax.experimental.pallas import tpu as pltpu
from jax.experimental.pallas import tpu_sc as plsc

N = 10000
NP = 10240
E = 320000
F_IN = 128
F_HID = 256
F = 128
NC = 2
NS = 16
NW = NC * NS
EPW = E // NW
CH = 100
CB = 100
RPB = NP // NS


def _agg_body(y_hbm, src_hbm, dst_hbm, zeros_hbm, out_hbm,
              srcs_v, dsts_v, rows_v, acc_sh, gsem, ssem):
    c = lax.axis_index("c")
    s = lax.axis_index("s")
    wid = c * NS + s

    pltpu.sync_copy(zeros_hbm.at[pl.ds(s * RPB, RPB)],
                    acc_sh.at[pl.ds(s * RPB, RPB)])

    pltpu.sync_copy(src_hbm.at[wid], srcs_v)
    pltpu.sync_copy(dst_hbm.at[wid], dsts_v)
    plsc.subcore_barrier()

    def chunk(j, carry):
        pltpu.async_copy(y_hbm.at[srcs_v.at[j]], rows_v, gsem).wait()
        pltpu.async_copy(rows_v, acc_sh.at[dsts_v.at[j]], ssem, add=True).wait()
        return carry

    lax.fori_loop(0, CH, chunk, 0, unroll=False)

    plsc.subcore_barrier()
    pltpu.sync_copy(acc_sh.at[pl.ds(s * RPB, RPB)],
                    out_hbm.at[c, pl.ds(s * RPB, RPB)])


@functools.cache
def _make_agg():
  return pl.kernel(
    _agg_body,
    out_type=jax.ShapeDtypeStruct((NC, NP, F), jnp.float32),
    mesh=plsc.VectorSubcoreMesh(core_axis_name="c", subcore_axis_name="s",
                                num_cores=NC, num_subcores=NS),
    scratch_types=[
        pltpu.VMEM((CH, CB), jnp.int32),
        pltpu.VMEM((CH, CB), jnp.int32),
        pltpu.VMEM((CB, F), jnp.float32),
        pltpu.VMEM_SHARED((NP, F), jnp.float32),
        pltpu.SemaphoreType.DMA,
        pltpu.SemaphoreType.DMA,
    ],
  )


_RB = 2048


def _tc1_body(degp, x, y1, dinvb):
    deg = degp[0] + degp[1] + 1.0
    dv = lax.rsqrt(deg)
    dinvb[...] = dv
    y1[...] = dv * x[...]


def _tc2_body(p, y1, dinvb, w1, b1, w2, y2):
    m = dinvb[...] * (p[0] + p[1] + y1[...])
    h = jax.nn.relu(jnp.dot(m, w1[...], preferred_element_type=jnp.float32)
                    + b1[...])
    z = jnp.dot(h, w2[...], preferred_element_type=jnp.float32)
    y2[...] = dinvb[...] * z


def _tc3_body(p, y2, dinvb, b2, out):
    a = dinvb[...] * (p[0] + p[1] + y2[...])
    out[...] = jax.nn.relu(a + b2[...])


def _row_specs(*widths):
    return [pl.BlockSpec((2, _RB, w) if three else (_RB, w),
                         (lambda i: (0, i, 0)) if three else (lambda i: (i, 0)))
            for three, w in widths]


_tc1 = pl.pallas_call(
    _tc1_body,
    grid=(NP // _RB,),
    in_specs=_row_specs((True, F), (False, F_IN)),
    out_specs=_row_specs((False, F_IN), (False, F)),
    out_shape=[jax.ShapeDtypeStruct((NP, F_IN), jnp.float32),
               jax.ShapeDtypeStruct((NP, F), jnp.float32)],
)

_tc2 = pl.pallas_call(
    _tc2_body,
    grid=(NP // _RB,),
    in_specs=_row_specs((True, F), (False, F), (False, F)) + [
        pl.BlockSpec((F_IN, F_HID), lambda i: (0, 0)),
        pl.BlockSpec((1, F_HID), lambda i: (0, 0)),
        pl.BlockSpec((F_HID, F), lambda i: (0, 0)),
    ],
    out_specs=_row_specs((False, F)),
    out_shape=[jax.ShapeDtypeStruct((NP, F), jnp.float32)],
)

_tc3 = pl.pallas_call(
    _tc3_body,
    grid=(NP // _RB,),
    in_specs=_row_specs((True, F), (False, F), (False, F)) + [
        pl.BlockSpec((1, F), lambda i: (0, 0)),
    ],
    out_specs=_row_specs((False, F)),
    out_shape=[jax.ShapeDtypeStruct((NP, F), jnp.float32)],
)


def kernel(x, edge_index, W1, b1, W2, b2):
    src = edge_index[0].astype(jnp.int32).reshape(NW, CH, CB)
    dst = edge_index[1].astype(jnp.int32).reshape(NW, CH, CB)
    zeros = jnp.zeros((NP, F), jnp.float32)
    ones = jnp.ones((NP, F), jnp.float32)
    xp = jnp.pad(x, ((0, NP - N), (0, 0)))
    _agg = _make_agg()

    degp = _agg(ones, src, dst, zeros)
    y1, dinvb = _tc1(degp, xp)
    p1 = _agg(y1, src, dst, zeros)
    [y2] = _tc2(p1, y1, dinvb, W1.astype(jnp.float32),
                b1.reshape(1, F_HID), W2.astype(jnp.float32))
    p2 = _agg(y2, src, dst, zeros)
    [out] = _tc3(p2, y2, dinvb, b2.reshape(1, F))
    return out[:N]


# speedup vs baseline: 37.2171x; 2.0971x over previous
import functools

import jax
import jax.numpy as jnp
from jax import lax
from jax.experimental import pallas as pl
from jax.experimental.pallas import tpu as pltpu
from jax.experimental.pallas import tpu_sc as plsc

N = 10000
NP = 10240
E = 320000
F_IN = 128
F_HID = 256
F = 128
FD = 16
NC = 2
NS = 16
NW = NC * NS
EPW = E // NW
CH = 100
CB = 100
RPB = NP // NS
NBUF = 2


def _agg_body(y_hbm, src_hbm, dst_hbm, out_hbm,
              srcs_v, dsts_v, rows_v, acc_sh, gsem, ssem):
    c = lax.axis_index("c")
    s = lax.axis_index("s")
    wid = c * NS + s

    pltpu.sync_copy(y_hbm.at[pl.ds(s * RPB, RPB)],
                    acc_sh.at[pl.ds(s * RPB, RPB)])

    pltpu.sync_copy(src_hbm.at[wid], srcs_v)
    pltpu.sync_copy(dst_hbm.at[wid], dsts_v)
    plsc.subcore_barrier()

    def g_start(j, k):
        pltpu.async_copy(y_hbm.at[srcs_v.at[j]], rows_v.at[k], gsem.at[k])

    def g_wait(j, k):
        pltpu.make_async_copy(y_hbm.at[srcs_v.at[j]], rows_v.at[k],
                              gsem.at[k]).wait()

    def s_start(j, k):
        pltpu.async_copy(rows_v.at[k], acc_sh.at[dsts_v.at[j]],
                         ssem, add=True)

    def s_wait(j, k):
        pltpu.make_async_copy(rows_v.at[k], acc_sh.at[dsts_v.at[j]],
                              ssem).wait()

    for k in range(NBUF):
        g_start(k, k)
    g_wait(0, 0)
    s_start(0, 0)

    def group(it, carry):
        for k in range(NBUF):
            j = 1 + it * NBUF + k
            s_wait(j - 1, k)
            g_start(j + NBUF - 1, k)
            g_wait(j, (k + 1) % NBUF)
            s_start(j, (k + 1) % NBUF)
        return carry

    lax.fori_loop(0, (CH - NBUF) // NBUF, group, 0, unroll=False)

    for k in range(NBUF - 1):
        j = CH - NBUF + 1 + k
        s_wait(j - 1, k)
        g_wait(j, (k + 1) % NBUF)
        s_start(j, (k + 1) % NBUF)
    s_wait(CH - 1, (CH - 1) % NBUF)

    plsc.subcore_barrier()
    pltpu.sync_copy(acc_sh.at[pl.ds(s * RPB, RPB)],
                    out_hbm.at[c, pl.ds(s * RPB, RPB)])


def _deg_body(dst_hbm, out_hbm, dsts_v, deg_v):
    c = lax.axis_index("c")
    s = lax.axis_index("s")
    wid = c * NS + s

    pltpu.sync_copy(dst_hbm.at[pl.ds(wid * EPW, EPW)], dsts_v)

    def zero(i, carry):
        deg_v[pl.ds(i * 16, 16)] = jnp.zeros((16,), jnp.float32)
        return carry

    lax.fori_loop(0, NP // 16, zero, 0, unroll=8)

    ones16 = jnp.ones((16,), jnp.float32)

    def count(j, carry):
        idx = dsts_v[pl.ds(j * 16, 16)]
        plsc.addupdate_scatter(deg_v, [idx], ones16)
        return carry

    lax.fori_loop(0, EPW // 16, count, 0, unroll=8)

    pltpu.sync_copy(deg_v, out_hbm.at[pl.ds(wid * NP, NP)])


@functools.cache
def _make_deg():
  return pl.kernel(
    _deg_body,
    out_type=jax.ShapeDtypeStruct((NW * NP,), jnp.float32),
    mesh=plsc.VectorSubcoreMesh(core_axis_name="c", subcore_axis_name="s",
                                num_cores=NC, num_subcores=NS),
    scratch_types=[
        pltpu.VMEM((EPW,), jnp.int32),
        pltpu.VMEM((NP,), jnp.float32),
    ],
    compiler_params=pltpu.CompilerParams(needs_layout_passes=False),
  )


@functools.cache
def _make_agg(fw):
  return pl.kernel(
    _agg_body,
    out_type=jax.ShapeDtypeStruct((NC, NP, fw), jnp.float32),
    mesh=plsc.VectorSubcoreMesh(core_axis_name="c", subcore_axis_name="s",
                                num_cores=NC, num_subcores=NS),
    scratch_types=[
        pltpu.VMEM((CH, CB), jnp.int32),
        pltpu.VMEM((CH, CB), jnp.int32),
        pltpu.VMEM((NBUF, CB, fw), jnp.float32),
        pltpu.VMEM_SHARED((NP, fw), jnp.float32),
        pltpu.SemaphoreType.DMA((NBUF,)),
        pltpu.SemaphoreType.DMA,
    ],
    compiler_params=pltpu.CompilerParams(use_tc_tiling_on_sc=False),
  )


_RB = 2048


def _tc1_body(degt, x, y1, dinvb):
    deg = lax.dot_general(degt[...], jnp.ones((NW, 1), jnp.float32),
                          (((0,), (0,)), ((), ())),
                          preferred_element_type=jnp.float32) + 1.0
    dv = lax.rsqrt(deg)
    dinvb[...] = jnp.broadcast_to(dv, (_RB, F))
    y1[...] = dv * x[...]


def _tc2_body(p, y1, dinvb, w1, b1, w2, y2):
    m = dinvb[...] * (p[0] + p[1] - y1[...])
    h = jax.nn.relu(jnp.dot(m, w1[...], preferred_element_type=jnp.float32)
                    + b1[...])
    z = jnp.dot(h, w2[...], preferred_element_type=jnp.float32)
    y2[...] = dinvb[...] * z


def _tc3_body(p, y2, dinvb, b2, out):
    a = dinvb[...] * (p[0] + p[1] - y2[...])
    out[...] = jax.nn.relu(a + b2[...])


def _row_specs(*widths):
    return [pl.BlockSpec((2, _RB, w) if three else (_RB, w),
                         (lambda i: (0, i, 0)) if three else (lambda i: (i, 0)))
            for three, w in widths]


_tc1 = pl.pallas_call(
    _tc1_body,
    grid=(NP // _RB,),
    in_specs=[pl.BlockSpec((NW, _RB), lambda i: (0, i))]
             + _row_specs((False, F_IN)),
    out_specs=_row_specs((False, F_IN), (False, F)),
    out_shape=[jax.ShapeDtypeStruct((NP, F_IN), jnp.float32),
               jax.ShapeDtypeStruct((NP, F), jnp.float32)],
)

_tc2 = pl.pallas_call(
    _tc2_body,
    grid=(NP // _RB,),
    in_specs=_row_specs((True, F), (False, F), (False, F)) + [
        pl.BlockSpec((F_IN, F_HID), lambda i: (0, 0)),
        pl.BlockSpec((1, F_HID), lambda i: (0, 0)),
        pl.BlockSpec((F_HID, F), lambda i: (0, 0)),
    ],
    out_specs=_row_specs((False, F)),
    out_shape=[jax.ShapeDtypeStruct((NP, F), jnp.float32)],
)

_tc3 = pl.pallas_call(
    _tc3_body,
    grid=(NP // _RB,),
    in_specs=_row_specs((True, F), (False, F), (False, F)) + [
        pl.BlockSpec((1, F), lambda i: (0, 0)),
    ],
    out_specs=_row_specs((False, F)),
    out_shape=[jax.ShapeDtypeStruct((NP, F), jnp.float32)],
)


def kernel(x, edge_index, W1, b1, W2, b2):
    src = edge_index[0].astype(jnp.int32).reshape(NW, CH, CB)
    dst = edge_index[1].astype(jnp.int32).reshape(NW, CH, CB)
    dstf = edge_index[1].astype(jnp.int32)
    xp = jnp.pad(x, ((0, NP - N), (0, 0)))

    degp = _make_deg()(dstf)
    y1, dinvb = _tc1(degp.reshape(NW, NP), xp)
    p1 = _make_agg(F)(y1, src, dst)
    [y2] = _tc2(p1, y1, dinvb, W1.astype(jnp.float32),
                b1.reshape(1, F_HID), W2.astype(jnp.float32))
    p2 = _make_agg(F)(y2, src, dst)
    [out] = _tc3(p2, y2, dinvb, b2.reshape(1, F))
    return out[:N]


# speedup vs baseline: 39.1907x; 1.0530x over previous
import functools

import jax
import jax.numpy as jnp
from jax import lax
from jax.experimental import pallas as pl
from jax.experimental.pallas import tpu as pltpu
from jax.experimental.pallas import tpu_sc as plsc

N = 10000
NP = 10240
E = 320000
F_IN = 128
F_HID = 256
F = 128
FD = 16
NC = 2
NS = 16
NW = NC * NS
EPW = E // NW
CH = 100
CB = 100
RPB = NP // NS
NBUF = 2


_ZR = 32


def _agg_body(y_hbm, src_hbm, dst_hbm, out_hbm,
              srcs_v, dsts_v, rows_v, zbuf_v, acc_sh, gsem, ssem):
    c = lax.axis_index("c")
    s = lax.axis_index("s")
    wid = c * NS + s

    z16 = jnp.zeros((16,), jnp.float32)

    def zero_row(i, carry):
        for k in range(F // 16):
            zbuf_v[i, pl.ds(k * 16, 16)] = z16.astype(zbuf_v.dtype)
        return carry

    lax.fori_loop(0, _ZR, zero_row, 0, unroll=4)

    def clear(i, carry):
        pltpu.sync_copy(zbuf_v, acc_sh.at[pl.ds(s * RPB + i * _ZR, _ZR)])
        return carry

    lax.fori_loop(0, RPB // _ZR, clear, 0, unroll=False)

    pltpu.sync_copy(src_hbm.at[wid], srcs_v)
    pltpu.sync_copy(dst_hbm.at[wid], dsts_v)
    plsc.subcore_barrier()

    def g_start(j, k):
        pltpu.async_copy(y_hbm.at[srcs_v.at[j]], rows_v.at[k], gsem.at[k])

    def g_wait(j, k):
        pltpu.make_async_copy(y_hbm.at[srcs_v.at[j]], rows_v.at[k],
                              gsem.at[k]).wait()

    def s_start(j, k):
        pltpu.async_copy(rows_v.at[k], acc_sh.at[dsts_v.at[j]],
                         ssem, add=True)

    def s_wait(j, k):
        pltpu.make_async_copy(rows_v.at[k], acc_sh.at[dsts_v.at[j]],
                              ssem).wait()

    for k in range(NBUF):
        g_start(k, k)
    g_wait(0, 0)
    s_start(0, 0)

    def group(it, carry):
        for k in range(NBUF):
            j = 1 + it * NBUF + k
            s_wait(j - 1, k)
            g_start(j + NBUF - 1, k)
            g_wait(j, (k + 1) % NBUF)
            s_start(j, (k + 1) % NBUF)
        return carry

    lax.fori_loop(0, (CH - NBUF) // NBUF, group, 0, unroll=False)

    for k in range(NBUF - 1):
        j = CH - NBUF + 1 + k
        s_wait(j - 1, k)
        g_wait(j, (k + 1) % NBUF)
        s_start(j, (k + 1) % NBUF)
    s_wait(CH - 1, (CH - 1) % NBUF)

    plsc.subcore_barrier()
    pltpu.sync_copy(acc_sh.at[pl.ds(s * RPB, RPB)],
                    out_hbm.at[c, pl.ds(s * RPB, RPB)])


def _deg_body(dst_hbm, out_hbm, dsts_v, deg_v):
    c = lax.axis_index("c")
    s = lax.axis_index("s")
    wid = c * NS + s

    pltpu.sync_copy(dst_hbm.at[pl.ds(wid * EPW, EPW)], dsts_v)

    def zero(i, carry):
        deg_v[pl.ds(i * 16, 16)] = jnp.zeros((16,), jnp.float32)
        return carry

    lax.fori_loop(0, NP // 16, zero, 0, unroll=8)

    ones16 = jnp.ones((16,), jnp.float32)

    def count(j, carry):
        idx = dsts_v[pl.ds(j * 16, 16)]
        plsc.addupdate_scatter(deg_v, [idx], ones16)
        return carry

    lax.fori_loop(0, EPW // 16, count, 0, unroll=8)

    pltpu.sync_copy(deg_v, out_hbm.at[pl.ds(wid * NP, NP)])


@functools.cache
def _make_deg():
  return pl.kernel(
    _deg_body,
    out_type=jax.ShapeDtypeStruct((NW * NP,), jnp.float32),
    mesh=plsc.VectorSubcoreMesh(core_axis_name="c", subcore_axis_name="s",
                                num_cores=NC, num_subcores=NS),
    scratch_types=[
        pltpu.VMEM((EPW,), jnp.int32),
        pltpu.VMEM((NP,), jnp.float32),
    ],
    compiler_params=pltpu.CompilerParams(needs_layout_passes=False),
  )


@functools.cache
def _make_agg(fw):
  return pl.kernel(
    _agg_body,
    out_type=jax.ShapeDtypeStruct((NC, NP, fw), jnp.bfloat16),
    mesh=plsc.VectorSubcoreMesh(core_axis_name="c", subcore_axis_name="s",
                                num_cores=NC, num_subcores=NS),
    scratch_types=[
        pltpu.VMEM((CH, CB), jnp.int32),
        pltpu.VMEM((CH, CB), jnp.int32),
        pltpu.VMEM((NBUF, CB, fw), jnp.bfloat16),
        pltpu.VMEM((_ZR, fw), jnp.bfloat16),
        pltpu.VMEM_SHARED((NP, fw), jnp.bfloat16),
        pltpu.SemaphoreType.DMA((NBUF,)),
        pltpu.SemaphoreType.DMA,
    ],
    compiler_params=pltpu.CompilerParams(use_tc_tiling_on_sc=False),
  )


_RB = 2048


def _tc1_body(degt, x, y1, dinvb):
    deg = lax.dot_general(degt[...], jnp.ones((NW, 1), jnp.float32),
                          (((0,), (0,)), ((), ())),
                          preferred_element_type=jnp.float32) + 1.0
    dv = lax.rsqrt(deg)
    dinvb[...] = jnp.broadcast_to(dv, (_RB, F))
    y1[...] = (dv * x[...]).astype(jnp.bfloat16)


def _tc2_body(p, y1, dinvb, w1, b1, w2, y2):
    pf = p[0].astype(jnp.float32) + p[1].astype(jnp.float32)
    m = dinvb[...] * (pf + y1[...].astype(jnp.float32))
    h = jax.nn.relu(jnp.dot(m, w1[...], preferred_element_type=jnp.float32)
                    + b1[...])
    z = jnp.dot(h, w2[...], preferred_element_type=jnp.float32)
    y2[...] = (dinvb[...] * z).astype(jnp.bfloat16)


def _tc3_body(p, y2, dinvb, b2, out):
    pf = p[0].astype(jnp.float32) + p[1].astype(jnp.float32)
    a = dinvb[...] * (pf + y2[...].astype(jnp.float32))
    out[...] = jax.nn.relu(a + b2[...])


def _row_specs(*widths):
    return [pl.BlockSpec((2, _RB, w) if three else (_RB, w),
                         (lambda i: (0, i, 0)) if three else (lambda i: (i, 0)))
            for three, w in widths]


_tc1 = pl.pallas_call(
    _tc1_body,
    grid=(NP // _RB,),
    in_specs=[pl.BlockSpec((NW, _RB), lambda i: (0, i))]
             + _row_specs((False, F_IN)),
    out_specs=_row_specs((False, F), (False, F_IN)),
    out_shape=[jax.ShapeDtypeStruct((NP, F), jnp.bfloat16),
               jax.ShapeDtypeStruct((NP, F_IN), jnp.float32)],
)

_tc2 = pl.pallas_call(
    _tc2_body,
    grid=(NP // _RB,),
    in_specs=_row_specs((True, F), (False, F), (False, F)) + [
        pl.BlockSpec((F_IN, F_HID), lambda i: (0, 0)),
        pl.BlockSpec((1, F_HID), lambda i: (0, 0)),
        pl.BlockSpec((F_HID, F), lambda i: (0, 0)),
    ],
    out_specs=_row_specs((False, F)),
    out_shape=[jax.ShapeDtypeStruct((NP, F), jnp.bfloat16)],
)

_tc3 = pl.pallas_call(
    _tc3_body,
    grid=(NP // _RB,),
    in_specs=_row_specs((True, F), (False, F), (False, F)) + [
        pl.BlockSpec((1, F), lambda i: (0, 0)),
    ],
    out_specs=_row_specs((False, F)),
    out_shape=[jax.ShapeDtypeStruct((NP, F), jnp.float32)],
)


def kernel(x, edge_index, W1, b1, W2, b2):
    src = edge_index[0].astype(jnp.int32).reshape(NW, CH, CB)
    dst = edge_index[1].astype(jnp.int32).reshape(NW, CH, CB)
    dstf = edge_index[1].astype(jnp.int32)
    xp = jnp.pad(x, ((0, NP - N), (0, 0)))

    degp = _make_deg()(dstf)
    y1, dinvb = _tc1(degp.reshape(NW, NP), xp)
    p1 = _make_agg(F)(y1, src, dst)
    [y2] = _tc2(p1, y1, dinvb, W1.astype(jnp.float32),
                b1.reshape(1, F_HID), W2.astype(jnp.float32))
    p2 = _make_agg(F)(y2, src, dst)
    [out] = _tc3(p2, y2, dinvb, b2.reshape(1, F))
    return out[:N]


# speedup vs baseline: 46.1397x; 1.1773x over previous
import functools

import jax
import jax.numpy as jnp
from jax import lax
from jax.experimental import pallas as pl
from jax.experimental.pallas import tpu as pltpu
from jax.experimental.pallas import tpu_sc as plsc

N = 10000
NP = 10240
E = 320000
F_IN = 128
F_HID = 256
F = 128
FD = 16
NC = 2
NS = 16
NW = NC * NS
EPW = E // NW
CH = 100
CB = 100
RPB = NP // NS
NBUF = 4


_ZR = 32


def _agg_body(y_hbm, src_hbm, dst_hbm, out_hbm,
              srcs_v, dsts_v, rows_v, zbuf_v, acc_sh, gsem, ssem):
    c = lax.axis_index("c")
    s = lax.axis_index("s")
    wid = c * NS + s

    z16 = jnp.zeros((16,), jnp.float32)

    def zero_row(i, carry):
        for k in range(F // 16):
            zbuf_v[i, pl.ds(k * 16, 16)] = z16.astype(zbuf_v.dtype)
        return carry

    lax.fori_loop(0, _ZR, zero_row, 0, unroll=4)

    def clear(i, carry):
        pltpu.sync_copy(zbuf_v, acc_sh.at[pl.ds(s * RPB + i * _ZR, _ZR)])
        return carry

    lax.fori_loop(0, RPB // _ZR, clear, 0, unroll=False)

    pltpu.sync_copy(src_hbm.at[wid], srcs_v)
    pltpu.sync_copy(dst_hbm.at[wid], dsts_v)
    plsc.subcore_barrier()

    def g_start(j, k):
        pltpu.async_copy(y_hbm.at[srcs_v.at[j]], rows_v.at[k], gsem.at[k])

    def g_wait(j, k):
        pltpu.make_async_copy(y_hbm.at[srcs_v.at[j]], rows_v.at[k],
                              gsem.at[k]).wait()

    def s_start(j, k):
        pltpu.async_copy(rows_v.at[k], acc_sh.at[dsts_v.at[j]],
                         ssem, add=True)

    def s_wait(j, k):
        pltpu.make_async_copy(rows_v.at[k], acc_sh.at[dsts_v.at[j]],
                              ssem).wait()

    for k in range(NBUF):
        g_start(k, k)
    g_wait(0, 0)
    s_start(0, 0)

    def group(it, carry):
        for k in range(NBUF):
            j = 1 + it * NBUF + k
            s_wait(j - 1, k)
            g_start(j + NBUF - 1, k)
            g_wait(j, (k + 1) % NBUF)
            s_start(j, (k + 1) % NBUF)
        return carry

    lax.fori_loop(0, (CH - NBUF) // NBUF, group, 0, unroll=False)

    for k in range(NBUF - 1):
        j = CH - NBUF + 1 + k
        s_wait(j - 1, k)
        g_wait(j, (k + 1) % NBUF)
        s_start(j, (k + 1) % NBUF)
    s_wait(CH - 1, (CH - 1) % NBUF)

    plsc.subcore_barrier()
    pltpu.sync_copy(acc_sh.at[pl.ds(s * RPB, RPB)],
                    out_hbm.at[c, pl.ds(s * RPB, RPB)])


def _deg_body(dst_hbm, out_hbm, dsts_v, deg_v):
    c = lax.axis_index("c")
    s = lax.axis_index("s")
    wid = c * NS + s

    pltpu.sync_copy(dst_hbm.at[pl.ds(wid * EPW, EPW)], dsts_v)

    def zero(i, carry):
        deg_v[pl.ds(i * 16, 16)] = jnp.zeros((16,), jnp.float32)
        return carry

    lax.fori_loop(0, NP // 16, zero, 0, unroll=8)

    ones16 = jnp.ones((16,), jnp.float32)

    def count(j, carry):
        idx = dsts_v[pl.ds(j * 16, 16)]
        plsc.addupdate_scatter(deg_v, [idx], ones16)
        return carry

    lax.fori_loop(0, EPW // 16, count, 0, unroll=8)

    pltpu.sync_copy(deg_v, out_hbm.at[pl.ds(wid * NP, NP)])


@functools.cache
def _make_deg():
  return pl.kernel(
    _deg_body,
    out_type=jax.ShapeDtypeStruct((NW * NP,), jnp.float32),
    mesh=plsc.VectorSubcoreMesh(core_axis_name="c", subcore_axis_name="s",
                                num_cores=NC, num_subcores=NS),
    scratch_types=[
        pltpu.VMEM((EPW,), jnp.int32),
        pltpu.VMEM((NP,), jnp.float32),
    ],
    compiler_params=pltpu.CompilerParams(needs_layout_passes=False),
  )


@functools.cache
def _make_agg(fw):
  return pl.kernel(
    _agg_body,
    out_type=jax.ShapeDtypeStruct((NC, NP, fw), jnp.bfloat16),
    mesh=plsc.VectorSubcoreMesh(core_axis_name="c", subcore_axis_name="s",
                                num_cores=NC, num_subcores=NS),
    scratch_types=[
        pltpu.VMEM((CH, CB), jnp.int32),
        pltpu.VMEM((CH, CB), jnp.int32),
        pltpu.VMEM((NBUF, CB, fw), jnp.bfloat16),
        pltpu.VMEM((_ZR, fw), jnp.bfloat16),
        pltpu.VMEM_SHARED((NP, fw), jnp.bfloat16),
        pltpu.SemaphoreType.DMA((NBUF,)),
        pltpu.SemaphoreType.DMA,
    ],
    compiler_params=pltpu.CompilerParams(use_tc_tiling_on_sc=False),
  )


_RB = 2048


def _tc1_body(degt, x, y1, dinvb):
    deg = lax.dot_general(degt[...], jnp.ones((NW, 1), jnp.float32),
                          (((0,), (0,)), ((), ())),
                          preferred_element_type=jnp.float32) + 1.0
    dv = lax.rsqrt(deg)
    dinvb[...] = jnp.broadcast_to(dv, (_RB, F))
    y1[...] = (dv * x[...]).astype(jnp.bfloat16)


def _tc2_body(p, y1, dinvb, w1, b1, w2, y2):
    pf = p[0].astype(jnp.float32) + p[1].astype(jnp.float32)
    m = dinvb[...] * (pf + y1[...].astype(jnp.float32))
    h = jax.nn.relu(jnp.dot(m, w1[...], preferred_element_type=jnp.float32)
                    + b1[...])
    z = jnp.dot(h, w2[...], preferred_element_type=jnp.float32)
    y2[...] = (dinvb[...] * z).astype(jnp.bfloat16)


def _tc3_body(p, y2, dinvb, b2, out):
    pf = p[0].astype(jnp.float32) + p[1].astype(jnp.float32)
    a = dinvb[...] * (pf + y2[...].astype(jnp.float32))
    out[...] = jax.nn.relu(a + b2[...])


def _row_specs(*widths):
    return [pl.BlockSpec((2, _RB, w) if three else (_RB, w),
                         (lambda i: (0, i, 0)) if three else (lambda i: (i, 0)))
            for three, w in widths]


_tc1 = pl.pallas_call(
    _tc1_body,
    grid=(NP // _RB,),
    in_specs=[pl.BlockSpec((NW, _RB), lambda i: (0, i))]
             + _row_specs((False, F_IN)),
    out_specs=_row_specs((False, F), (False, F_IN)),
    out_shape=[jax.ShapeDtypeStruct((NP, F), jnp.bfloat16),
               jax.ShapeDtypeStruct((NP, F_IN), jnp.float32)],
)

_tc2 = pl.pallas_call(
    _tc2_body,
    grid=(NP // _RB,),
    in_specs=_row_specs((True, F), (False, F), (False, F)) + [
        pl.BlockSpec((F_IN, F_HID), lambda i: (0, 0)),
        pl.BlockSpec((1, F_HID), lambda i: (0, 0)),
        pl.BlockSpec((F_HID, F), lambda i: (0, 0)),
    ],
    out_specs=_row_specs((False, F)),
    out_shape=[jax.ShapeDtypeStruct((NP, F), jnp.bfloat16)],
)

_tc3 = pl.pallas_call(
    _tc3_body,
    grid=(NP // _RB,),
    in_specs=_row_specs((True, F), (False, F), (False, F)) + [
        pl.BlockSpec((1, F), lambda i: (0, 0)),
    ],
    out_specs=_row_specs((False, F)),
    out_shape=[jax.ShapeDtypeStruct((NP, F), jnp.float32)],
)


def kernel(x, edge_index, W1, b1, W2, b2):
    src = edge_index[0].astype(jnp.int32).reshape(NW, CH, CB)
    dst = edge_index[1].astype(jnp.int32).reshape(NW, CH, CB)
    dstf = edge_index[1].astype(jnp.int32)
    xp = jnp.pad(x, ((0, NP - N), (0, 0)))

    degp = _make_deg()(dstf)
    y1, dinvb = _tc1(degp.reshape(NW, NP), xp)
    p1 = _make_agg(F)(y1, src, dst)
    [y2] = _tc2(p1, y1, dinvb, W1.astype(jnp.float32),
                b1.reshape(1, F_HID), W2.astype(jnp.float32))
    p2 = _make_agg(F)(y2, src, dst)
    [out] = _tc3(p2, y2, dinvb, b2.reshape(1, F))
    return out[:N]


# speedup vs baseline: 47.5027x; 1.0295x over previous
import functools

import jax
import jax.numpy as jnp
from jax import lax
from jax.experimental import pallas as pl
from jax.experimental.pallas import tpu as pltpu
from jax.experimental.pallas import tpu_sc as plsc

N = 10000
NP = 10240
E = 320000
F_IN = 128
F_HID = 256
F = 128
FD = 16
NC = 2
NS = 16
NW = NC * NS
EPW = E // NW
CH = 100
CB = 100
RPB = NP // NS
NBUF = 4


_ZR = 32


def _agg_body(y_hbm, src_hbm, dst_hbm, out_hbm,
              srcs_v, dsts_v, rows_v, zbuf_v, acc_sh, gsem, ssem):
    c = lax.axis_index("c")
    s = lax.axis_index("s")
    wid = c * NS + s

    z16 = jnp.zeros((16,), jnp.float32)

    def zero_row(i, carry):
        for k in range(F // 16):
            zbuf_v[i, pl.ds(k * 16, 16)] = z16.astype(zbuf_v.dtype)
        return carry

    lax.fori_loop(0, _ZR, zero_row, 0, unroll=4)

    def clear(i, carry):
        pltpu.sync_copy(zbuf_v, acc_sh.at[pl.ds(s * RPB + i * _ZR, _ZR)])
        return carry

    lax.fori_loop(0, RPB // _ZR, clear, 0, unroll=False)

    pltpu.sync_copy(src_hbm.at[wid], srcs_v)
    pltpu.sync_copy(dst_hbm.at[wid], dsts_v)
    plsc.subcore_barrier()

    def g_start(j, k):
        pltpu.async_copy(y_hbm.at[srcs_v.at[j]], rows_v.at[k], gsem.at[k])

    def g_wait(j, k):
        pltpu.make_async_copy(y_hbm.at[srcs_v.at[j]], rows_v.at[k],
                              gsem.at[k]).wait()

    def s_start(j, k):
        pltpu.async_copy(rows_v.at[k], acc_sh.at[dsts_v.at[j]],
                         ssem, add=True)

    def s_wait(j, k):
        pltpu.make_async_copy(rows_v.at[k], acc_sh.at[dsts_v.at[j]],
                              ssem).wait()

    for k in range(NBUF):
        g_start(k, k)
    g_wait(0, 0)
    s_start(0, 0)

    def group(it, carry):
        for k in range(NBUF):
            j = 1 + it * NBUF + k
            s_wait(j - 1, k)
            g_start(j + NBUF - 1, k)
            g_wait(j, (k + 1) % NBUF)
            s_start(j, (k + 1) % NBUF)
        return carry

    lax.fori_loop(0, (CH - NBUF) // NBUF, group, 0, unroll=False)

    for k in range(NBUF - 1):
        j = CH - NBUF + 1 + k
        s_wait(j - 1, k)
        g_wait(j, (k + 1) % NBUF)
        s_start(j, (k + 1) % NBUF)
    s_wait(CH - 1, (CH - 1) % NBUF)

    plsc.subcore_barrier()
    pltpu.sync_copy(acc_sh.at[pl.ds(s * RPB, RPB)],
                    out_hbm.at[c, pl.ds(s * RPB, RPB)])


def _deg_body(dst_hbm, out_hbm, dsts_v, deg_v):
    c = lax.axis_index("c")
    s = lax.axis_index("s")
    wid = c * NS + s

    pltpu.sync_copy(dst_hbm.at[pl.ds(wid * EPW, EPW)], dsts_v)

    def zero(i, carry):
        deg_v[pl.ds(i * 16, 16)] = jnp.zeros((16,), jnp.float32)
        return carry

    lax.fori_loop(0, NP // 16, zero, 0, unroll=8)

    ones16 = jnp.ones((16,), jnp.float32)

    def count(j, carry):
        idx = dsts_v[pl.ds(j * 16, 16)]
        plsc.addupdate_scatter(deg_v, [idx], ones16)
        return carry

    lax.fori_loop(0, EPW // 16, count, 0, unroll=8)

    pltpu.sync_copy(deg_v, out_hbm.at[pl.ds(wid * NP, NP)])


@functools.cache
def _make_deg():
  return pl.kernel(
    _deg_body,
    out_type=jax.ShapeDtypeStruct((NW * NP,), jnp.float32),
    mesh=plsc.VectorSubcoreMesh(core_axis_name="c", subcore_axis_name="s",
                                num_cores=NC, num_subcores=NS),
    scratch_types=[
        pltpu.VMEM((EPW,), jnp.int32),
        pltpu.VMEM((NP,), jnp.float32),
    ],
    compiler_params=pltpu.CompilerParams(needs_layout_passes=False),
  )


@functools.cache
def _make_agg(fw):
  return pl.kernel(
    _agg_body,
    out_type=jax.ShapeDtypeStruct((NC, NP, fw), jnp.bfloat16),
    mesh=plsc.VectorSubcoreMesh(core_axis_name="c", subcore_axis_name="s",
                                num_cores=NC, num_subcores=NS),
    scratch_types=[
        pltpu.VMEM((CH, CB), jnp.int32),
        pltpu.VMEM((CH, CB), jnp.int32),
        pltpu.VMEM((NBUF, CB, fw), jnp.bfloat16),
        pltpu.VMEM((_ZR, fw), jnp.bfloat16),
        pltpu.VMEM_SHARED((NP, fw), jnp.bfloat16),
        pltpu.SemaphoreType.DMA((NBUF,)),
        pltpu.SemaphoreType.DMA,
    ],
    compiler_params=pltpu.CompilerParams(use_tc_tiling_on_sc=False),
  )


_RB = 2048


def _tc1_body(degt, x, y1, dinvb):
    deg = lax.dot_general(degt[...], jnp.ones((NW, 1), jnp.float32),
                          (((0,), (0,)), ((), ())),
                          preferred_element_type=jnp.float32) + 1.0
    dv = lax.rsqrt(deg)
    dinvb[...] = jnp.broadcast_to(dv, (_RB, F))
    y1[...] = (dv * x[...]).astype(jnp.bfloat16)


def _tc2_body(p, y1, dinvb, w1, b1, w2, y2):
    pf = p[0].astype(jnp.float32) + p[1].astype(jnp.float32)
    m = dinvb[...] * (pf + y1[...].astype(jnp.float32))
    h = jax.nn.relu(jnp.dot(m.astype(jnp.bfloat16), w1[...],
                            preferred_element_type=jnp.float32) + b1[...])
    z = jnp.dot(h.astype(jnp.bfloat16), w2[...],
                preferred_element_type=jnp.float32)
    y2[...] = (dinvb[...] * z).astype(jnp.bfloat16)


def _tc3_body(p, y2, dinvb, b2, out):
    pf = p[0].astype(jnp.float32) + p[1].astype(jnp.float32)
    a = dinvb[...] * (pf + y2[...].astype(jnp.float32))
    out[...] = jax.nn.relu(a + b2[...])


def _row_specs(*widths):
    return [pl.BlockSpec((2, _RB, w) if three else (_RB, w),
                         (lambda i: (0, i, 0)) if three else (lambda i: (i, 0)))
            for three, w in widths]


_tc1 = pl.pallas_call(
    _tc1_body,
    grid=(NP // _RB,),
    in_specs=[pl.BlockSpec((NW, _RB), lambda i: (0, i))]
             + _row_specs((False, F_IN)),
    out_specs=_row_specs((False, F), (False, F_IN)),
    out_shape=[jax.ShapeDtypeStruct((NP, F), jnp.bfloat16),
               jax.ShapeDtypeStruct((NP, F_IN), jnp.float32)],
)

_tc2 = pl.pallas_call(
    _tc2_body,
    grid=(NP // _RB,),
    in_specs=_row_specs((True, F), (False, F), (False, F)) + [
        pl.BlockSpec((F_IN, F_HID), lambda i: (0, 0)),
        pl.BlockSpec((1, F_HID), lambda i: (0, 0)),
        pl.BlockSpec((F_HID, F), lambda i: (0, 0)),
    ],
    out_specs=_row_specs((False, F)),
    out_shape=[jax.ShapeDtypeStruct((NP, F), jnp.bfloat16)],
)

_tc3 = pl.pallas_call(
    _tc3_body,
    grid=(NP // _RB,),
    in_specs=_row_specs((True, F), (False, F), (False, F)) + [
        pl.BlockSpec((1, F), lambda i: (0, 0)),
    ],
    out_specs=_row_specs((False, F)),
    out_shape=[jax.ShapeDtypeStruct((NP, F), jnp.float32)],
)


def kernel(x, edge_index, W1, b1, W2, b2):
    src = edge_index[0].astype(jnp.int32).reshape(NW, CH, CB)
    dst = edge_index[1].astype(jnp.int32).reshape(NW, CH, CB)
    dstf = edge_index[1].astype(jnp.int32)
    degp = _make_deg()(dstf)
    y1, dinvb = _tc1(degp.reshape(NW, NP), x)
    p1 = _make_agg(F)(y1, src, dst)
    [y2] = _tc2(p1, y1, dinvb, W1.astype(jnp.bfloat16),
                b1.reshape(1, F_HID), W2.astype(jnp.bfloat16))
    p2 = _make_agg(F)(y2, src, dst)
    [out] = _tc3(p2, y2, dinvb, b2.reshape(1, F))
    return out[:N]
